# baseline (speedup 1.0000x reference)
import functools

import jax
import jax.numpy as jnp
from jax import lax
from jax.experimental import pallas as pl
from jax.experimental.pallas import tpu as pltpu
from jax.experimental.pallas import tpu_sc as plsc

_B, _T = 4096, 200
_V, _D = 100000, 32
_NC, _NS, _L = 2, 16, 16
_NW = _NC * _NS
_RPW = _B // _NW
_NCHUNK = (_T + _L - 1) // _L
_TAIL = _T - (_NCHUNK - 1) * _L


_VB = 2048
_VH = 53248
_NVB = _VH // _VB


def _round_bf16_bits(x):
    bits = jax.lax.bitcast_convert_type(x, jnp.int32)
    return bits + 0x8000


def _table_body(embt_lo_ref, embt_hi_ref, w_ref, b_ref, out_ref):
    w = w_ref[...]
    s_lo = (jnp.sum(embt_lo_ref[...] * w, axis=0) + b_ref[0, 0]) * (1.0 / _T)
    s_hi = (jnp.sum(embt_hi_ref[...] * w, axis=0) + b_ref[0, 0]) * (1.0 / _T)
    lo = jax.lax.shift_right_logical(_round_bf16_bits(s_lo), 16)
    hi = jnp.bitwise_and(_round_bf16_bits(s_hi), jnp.int32(-65536))
    out_ref[...] = jnp.bitwise_or(hi, lo)


_table_call = pl.pallas_call(
    _table_body,
    grid=(_NVB,),
    in_specs=[
        pl.BlockSpec((_D, _VB), lambda i: (0, i)),
        pl.BlockSpec((_D, _VB), lambda i: (0, jnp.minimum(i + _NVB, (_V - 1) // _VB))),
        pl.BlockSpec((_D, 1), lambda i: (0, 0)),
        pl.BlockSpec((1, 1), lambda i: (0, 0)),
    ],
    out_specs=pl.BlockSpec((_VB,), lambda i: (i,)),
    out_shape=jax.ShapeDtypeStruct((_VH,), jnp.int32),
)

_mesh = plsc.VectorSubcoreMesh(
    core_axis_name="c", subcore_axis_name="s", num_cores=_NC, num_subcores=_NS
)


_NG = _RPW // _L


@functools.partial(
    pl.kernel,
    out_type=jax.ShapeDtypeStruct((_B,), jnp.float32),
    mesh=_mesh,
    compiler_params=pltpu.CompilerParams(needs_layout_passes=False),
    scratch_types=[
        pltpu.VMEM((_VH,), jnp.int32),
        pltpu.VMEM((_T, _RPW), jnp.int32),
        pltpu.VMEM((_RPW,), jnp.float32),
        pltpu.SemaphoreType.DMA,
        pltpu.SemaphoreType.DMA,
    ],
)
def _sc_pool(s_hbm, idst_hbm, out_hbm, table_v, ids_v, out_v, sem_t, sem_i):
    wid = lax.axis_index("s") * _NC + lax.axis_index("c")
    col0 = pl.multiple_of(wid * _RPW, 8)
    h_t = pltpu.async_copy(s_hbm, table_v, sem_t)
    h_i = pltpu.async_copy(idst_hbm.at[:, pl.ds(col0, _RPW)], ids_v, sem_i)
    h_i.wait()
    h_t.wait()

    def token_body(t, accs):
        new = []
        for g in range(_NG):
            idx = ids_v[t, pl.ds(g * _L, _L)]
            is_hi = idx >= _VH
            word = idx - jnp.where(is_hi, jnp.int32(_VH), jnp.int32(0))
            val32 = plsc.load_gather(table_v, [word])
            bits = jnp.where(
                is_hi,
                jnp.bitwise_and(val32, jnp.int32(-65536)),
                jax.lax.shift_left(val32, 16),
            )
            new.append(accs[g] + plsc.bitcast(bits, jnp.float32))
        return tuple(new)

    accs = lax.fori_loop(
        0, _T, token_body, tuple(jnp.zeros((_L,), jnp.float32) for _ in range(_NG))
    )
    for g in range(_NG):
        out_v[pl.ds(g * _L, _L)] = accs[g]
    pltpu.sync_copy(out_v, out_hbm.at[pl.ds(col0, _RPW)])


def kernel(input_ids, emb, W, b):
    ids_t = input_ids.T.astype(jnp.int32)
    embt = emb.T
    s = _table_call(embt, embt, W, b.reshape(1, 1))
    return _sc_pool(s, ids_t)


# speedup vs baseline: 1.2001x; 1.2001x over previous
import functools

import jax
import jax.numpy as jnp
from jax import lax
from jax.experimental import pallas as pl
from jax.experimental.pallas import tpu as pltpu
from jax.experimental.pallas import tpu_sc as plsc

_B, _T = 4096, 200
_V, _D = 100000, 32
_NC, _NS, _L = 2, 16, 16
_NW = _NC * _NS
_RPW = _B // _NW
_NCHUNK = (_T + _L - 1) // _L
_TAIL = _T - (_NCHUNK - 1) * _L


_VB = 8192
_VH = 57344
_NVB = _VH // _VB


def _round_bf16_bits(x):
    bits = jax.lax.bitcast_convert_type(x, jnp.int32)
    return bits + 0x8000


def _table_body(embt_lo_ref, embt_hi_ref, w_ref, b_ref, out_ref):
    w = w_ref[...]
    s_lo = (jnp.sum(embt_lo_ref[...] * w, axis=0) + b_ref[0, 0]) * (1.0 / _T)
    s_hi = (jnp.sum(embt_hi_ref[...] * w, axis=0) + b_ref[0, 0]) * (1.0 / _T)
    lo = jax.lax.shift_right_logical(_round_bf16_bits(s_lo), 16)
    hi = jnp.bitwise_and(_round_bf16_bits(s_hi), jnp.int32(-65536))
    out_ref[...] = jnp.bitwise_or(hi, lo)


_table_call = pl.pallas_call(
    _table_body,
    grid=(_NVB,),
    in_specs=[
        pl.BlockSpec((_D, _VB), lambda i: (0, i)),
        pl.BlockSpec((_D, _VB), lambda i: (0, jnp.minimum(i + _NVB, (_V - 1) // _VB))),
        pl.BlockSpec((_D, 1), lambda i: (0, 0)),
        pl.BlockSpec((1, 1), lambda i: (0, 0)),
    ],
    out_specs=pl.BlockSpec((_VB,), lambda i: (i,)),
    out_shape=jax.ShapeDtypeStruct((_VH,), jnp.int32),
)

_mesh = plsc.VectorSubcoreMesh(
    core_axis_name="c", subcore_axis_name="s", num_cores=_NC, num_subcores=_NS
)


_NG = _RPW // _L


@functools.partial(
    pl.kernel,
    out_type=jax.ShapeDtypeStruct((_B,), jnp.float32),
    mesh=_mesh,
    compiler_params=pltpu.CompilerParams(needs_layout_passes=False),
    scratch_types=[
        pltpu.VMEM((_VH,), jnp.int32),
        pltpu.VMEM((_T, _RPW), jnp.int32),
        pltpu.VMEM((_RPW,), jnp.float32),
        pltpu.SemaphoreType.DMA,
        pltpu.SemaphoreType.DMA,
    ],
)
def _sc_pool(s_hbm, idst_hbm, out_hbm, table_v, ids_v, out_v, sem_t, sem_i):
    wid = lax.axis_index("s") * _NC + lax.axis_index("c")
    col0 = pl.multiple_of(wid * _RPW, 8)
    h_t = pltpu.async_copy(s_hbm, table_v, sem_t)
    h_i = pltpu.async_copy(idst_hbm.at[:, pl.ds(col0, _RPW)], ids_v, sem_i)
    h_i.wait()
    h_t.wait()

    def token_body(t, accs):
        new = []
        for g in range(_NG):
            idx = ids_v[t, pl.ds(g * _L, _L)]
            is_hi = idx >= _VH
            word = idx - jnp.where(is_hi, jnp.int32(_VH), jnp.int32(0))
            val32 = plsc.load_gather(table_v, [word])
            bits = jnp.where(
                is_hi,
                jnp.bitwise_and(val32, jnp.int32(-65536)),
                jax.lax.shift_left(val32, 16),
            )
            new.append(accs[g] + plsc.bitcast(bits, jnp.float32))
        return tuple(new)

    accs = lax.fori_loop(
        0, _T, token_body, tuple(jnp.zeros((_L,), jnp.float32) for _ in range(_NG))
    )
    for g in range(_NG):
        out_v[pl.ds(g * _L, _L)] = accs[g]
    pltpu.sync_copy(out_v, out_hbm.at[pl.ds(col0, _RPW)])


def kernel(input_ids, emb, W, b):
    ids_t = input_ids.T.astype(jnp.int32)
    embt = emb.T
    s = _table_call(embt, embt, W, b.reshape(1, 1))
    return _sc_pool(s, ids_t)


# speedup vs baseline: 1.2504x; 1.0419x over previous
import functools

import jax
import jax.numpy as jnp
from jax import lax
from jax.experimental import pallas as pl
from jax.experimental.pallas import tpu as pltpu
from jax.experimental.pallas import tpu_sc as plsc

_B, _T = 4096, 200
_V, _D = 100000, 32
_NC, _NS, _L = 2, 16, 16
_NW = _NC * _NS
_RPW = _B // _NW
_NCHUNK = (_T + _L - 1) // _L
_TAIL = _T - (_NCHUNK - 1) * _L


_VB = 16384
_VH = 65536
_NVB = _VH // _VB


def _round_bf16_bits(x):
    bits = jax.lax.bitcast_convert_type(x, jnp.int32)
    return bits + 0x8000


def _table_body(embt_lo_ref, embt_hi_ref, w_ref, b_ref, out_ref):
    w = w_ref[...]
    s_lo = (jnp.sum(embt_lo_ref[...] * w, axis=0) + b_ref[0, 0]) * (1.0 / _T)
    s_hi = (jnp.sum(embt_hi_ref[...] * w, axis=0) + b_ref[0, 0]) * (1.0 / _T)
    lo = jax.lax.shift_right_logical(_round_bf16_bits(s_lo), 16)
    hi = jnp.bitwise_and(_round_bf16_bits(s_hi), jnp.int32(-65536))
    out_ref[...] = jnp.bitwise_or(hi, lo)


_table_call = pl.pallas_call(
    _table_body,
    grid=(_NVB,),
    in_specs=[
        pl.BlockSpec((_D, _VB), lambda i: (0, i)),
        pl.BlockSpec((_D, _VB), lambda i: (0, jnp.minimum(i + _NVB, (_V - 1) // _VB))),
        pl.BlockSpec((_D, 1), lambda i: (0, 0)),
        pl.BlockSpec((1, 1), lambda i: (0, 0)),
    ],
    out_specs=pl.BlockSpec((_VB,), lambda i: (i,)),
    out_shape=jax.ShapeDtypeStruct((_VH,), jnp.int32),
)

_mesh = plsc.VectorSubcoreMesh(
    core_axis_name="c", subcore_axis_name="s", num_cores=_NC, num_subcores=_NS
)


_NG = _RPW // _L


@functools.partial(
    pl.kernel,
    out_type=jax.ShapeDtypeStruct((_B,), jnp.float32),
    mesh=_mesh,
    compiler_params=pltpu.CompilerParams(needs_layout_passes=False),
    scratch_types=[
        pltpu.VMEM((_VH,), jnp.int32),
        pltpu.VMEM((_T, _RPW), jnp.int32),
        pltpu.VMEM((_RPW,), jnp.float32),
        pltpu.SemaphoreType.DMA,
        pltpu.SemaphoreType.DMA,
    ],
)
def _sc_pool(s_hbm, idst_hbm, out_hbm, table_v, ids_v, out_v, sem_t, sem_i):
    wid = lax.axis_index("s") * _NC + lax.axis_index("c")
    col0 = pl.multiple_of(wid * _RPW, 8)
    h_t = pltpu.async_copy(s_hbm, table_v, sem_t)
    h_i = pltpu.async_copy(idst_hbm.at[:, pl.ds(col0, _RPW)], ids_v, sem_i)
    h_i.wait()
    h_t.wait()

    def token_body(t, accs):
        new = []
        for g in range(_NG):
            idx = ids_v[t, pl.ds(g * _L, _L)]
            word = jnp.bitwise_and(idx, jnp.int32(0xFFFF))
            shamt = jnp.bitwise_and(
                jnp.bitwise_not(jax.lax.shift_right_logical(idx, 12)),
                jnp.int32(16),
            )
            val32 = plsc.load_gather(table_v, [word])
            bits = jnp.bitwise_and(
                jax.lax.shift_left(val32, shamt), jnp.int32(-65536)
            )
            new.append(accs[g] + plsc.bitcast(bits, jnp.float32))
        return tuple(new)

    accs = lax.fori_loop(
        0, _T, token_body, tuple(jnp.zeros((_L,), jnp.float32) for _ in range(_NG))
    )
    for g in range(_NG):
        out_v[pl.ds(g * _L, _L)] = accs[g]
    pltpu.sync_copy(out_v, out_hbm.at[pl.ds(col0, _RPW)])


def kernel(input_ids, emb, W, b):
    ids_t = input_ids.T.astype(jnp.int32)
    embt = emb.T
    s = _table_call(embt, embt, W, b.reshape(1, 1))
    return _sc_pool(s, ids_t)


# speedup vs baseline: 1.2870x; 1.0293x over previous
import functools

import jax
import jax.numpy as jnp
from jax import lax
from jax.experimental import pallas as pl
from jax.experimental.pallas import tpu as pltpu
from jax.experimental.pallas import tpu_sc as plsc

_B, _T = 4096, 200
_V, _D = 100000, 32
_NC, _NS, _L = 2, 16, 16
_NW = _NC * _NS
_RPW = _B // _NW
_NCHUNK = (_T + _L - 1) // _L
_TAIL = _T - (_NCHUNK - 1) * _L


_VB = 16384
_VH = 65536
_NVB = _VH // _VB


def _round_bf16_bits(x):
    bits = jax.lax.bitcast_convert_type(x, jnp.int32)
    return bits + 0x8000


def _table_body(embt_lo_ref, embt_hi_ref, w_ref, b_ref, out_ref):
    w = w_ref[...].T
    s_lo = (jnp.sum(embt_lo_ref[...] * w, axis=0) + b_ref[0, 0]) * (1.0 / _T)
    s_hi = (jnp.sum(embt_hi_ref[...] * w, axis=0) + b_ref[0, 0]) * (1.0 / _T)
    lo = jax.lax.shift_right_logical(_round_bf16_bits(s_lo), 16)
    hi = jnp.bitwise_and(_round_bf16_bits(s_hi), jnp.int32(-65536))
    out_ref[...] = jnp.bitwise_or(hi, lo)


_table_call = pl.pallas_call(
    _table_body,
    grid=(_NVB,),
    in_specs=[
        pl.BlockSpec((_D, _VB), lambda i: (0, i)),
        pl.BlockSpec((_D, _VB), lambda i: (0, jnp.minimum(i + _NVB, (_V - 1) // _VB))),
        pl.BlockSpec((1, _D), lambda i: (0, 0)),
        pl.BlockSpec((1, 1), lambda i: (0, 0)),
    ],
    out_specs=pl.BlockSpec((_VB,), lambda i: (i,)),
    out_shape=jax.ShapeDtypeStruct((_VH,), jnp.int32),
)

_mesh = plsc.VectorSubcoreMesh(
    core_axis_name="c", subcore_axis_name="s", num_cores=_NC, num_subcores=_NS
)


_NG = _RPW // _L


@functools.partial(
    pl.kernel,
    out_type=jax.ShapeDtypeStruct((_B,), jnp.float32),
    mesh=_mesh,
    compiler_params=pltpu.CompilerParams(needs_layout_passes=False),
    scratch_types=[
        pltpu.VMEM((_VH,), jnp.int32),
        pltpu.VMEM((_T, _RPW), jnp.int32),
        pltpu.VMEM((_RPW,), jnp.float32),
        pltpu.SemaphoreType.DMA,
        pltpu.SemaphoreType.DMA,
    ],
)
def _sc_pool(s_hbm, idst_hbm, out_hbm, table_v, ids_v, out_v, sem_t, sem_i):
    wid = lax.axis_index("s") * _NC + lax.axis_index("c")
    col0 = pl.multiple_of(wid * _RPW, 8)
    h_t = pltpu.async_copy(s_hbm, table_v, sem_t)
    h_i = pltpu.async_copy(idst_hbm.at[:, pl.ds(col0, _RPW)], ids_v, sem_i)
    h_i.wait()
    h_t.wait()

    def token_body(t, accs):
        new = []
        for g in range(_NG):
            idx = ids_v[t, pl.ds(g * _L, _L)]
            word = jnp.bitwise_and(idx, jnp.int32(0xFFFF))
            shamt = jnp.bitwise_and(
                jnp.bitwise_not(jax.lax.shift_right_logical(idx, 12)),
                jnp.int32(16),
            )
            val32 = plsc.load_gather(table_v, [word])
            bits = jnp.bitwise_and(
                jax.lax.shift_left(val32, shamt), jnp.int32(-65536)
            )
            new.append(accs[g] + plsc.bitcast(bits, jnp.float32))
        return tuple(new)

    accs = lax.fori_loop(
        0, _T, token_body, tuple(jnp.zeros((_L,), jnp.float32) for _ in range(_NG))
    )
    for g in range(_NG):
        out_v[pl.ds(g * _L, _L)] = accs[g]
    pltpu.sync_copy(out_v, out_hbm.at[pl.ds(col0, _RPW)])


def kernel(input_ids, emb, W, b):
    ids_t = input_ids.T.astype(jnp.int32)
    embt = emb.T
    s = _table_call(embt, embt, W.T, b.reshape(1, 1))
    return _sc_pool(s, ids_t)
